# baseline (speedup 1.0000x reference)
import functools

import jax
import jax.numpy as jnp
from jax import lax
from jax.experimental import pallas as pl
from jax.experimental.pallas import tpu as pltpu
from jax.experimental.pallas import tpu_sc as plsc

NC = 2
NS = 16
LANES = 16
NW = NC * NS

D = 64
R = 512
GROUP = 128
G = R // GROUP


def _sc_embed(n_rows, n_scores, words_emb, scores_emb, widx2d, sidx_flat):
    per_w = n_rows // NW
    n_chunks = per_w // R

    mesh = plsc.VectorSubcoreMesh(
        core_axis_name="c", subcore_axis_name="s",
        num_cores=NC, num_subcores=NS,
    )

    @functools.partial(
        pl.kernel,
        out_type=jax.ShapeDtypeStruct((n_rows, D), jnp.float32),
        mesh=mesh,
        scratch_types=[
            pltpu.VMEM((n_scores, D), jnp.float32),
            pltpu.VMEM((G, GROUP), jnp.int32),
            pltpu.VMEM((R,), jnp.int32),
            pltpu.VMEM((R, D), jnp.float32),
            pltpu.SemaphoreType.DMA,
        ],
        compiler_params=pltpu.CompilerParams(
            needs_layout_passes=False, use_tc_tiling_on_sc=False),
    )
    def k(words_hbm, stab_hbm, widx_hbm, sidx_hbm, out_hbm,
          stab_v, widx_v, sidx_v, rows_v, sem):
        wid = lax.axis_index("s") * NC + lax.axis_index("c")
        base_row = wid * per_w
        base_grp = wid * (per_w // GROUP)

        pltpu.sync_copy(stab_hbm, stab_v)

        iota = lax.iota(jnp.int32, LANES)

        def chunk_body(c, carry):
            row0 = base_row + c * R
            grp0 = base_grp + c * G
            pltpu.sync_copy(widx_hbm.at[pl.ds(grp0, G)], widx_v)
            pltpu.sync_copy(sidx_hbm.at[pl.ds(row0, R)], sidx_v)
            descs = [
                pltpu.async_copy(
                    words_hbm.at[widx_v.at[j]],
                    rows_v.at[pl.ds(j * GROUP, GROUP)],
                    sem,
                )
                for j in range(G)
            ]
            for d in descs:
                d.wait()

            def grp_body(g, carry2):
                sidx = sidx_v[pl.ds(g * LANES, LANES)]
                rowidx = g * LANES + iota
                for col in range(D):
                    cvec = jnp.full((LANES,), col, jnp.int32)
                    sval = plsc.load_gather(stab_v, [sidx, cvec])
                    wval = plsc.load_gather(rows_v, [rowidx, cvec])
                    plsc.store_scatter(rows_v, [rowidx, cvec], wval + sval)
                return carry2

            lax.fori_loop(0, R // LANES, grp_body, 0, unroll=False)
            pltpu.sync_copy(rows_v, out_hbm.at[pl.ds(row0, R)])
            return carry

        lax.fori_loop(0, n_chunks, chunk_body, 0, unroll=False)

    return k(words_emb, scores_emb, widx2d, sidx_flat)


def kernel(input_ids, scores_ids, words_emb, scores_emb):
    b, l = input_ids.shape
    n = b * l
    widx2d = input_ids.astype(jnp.int32).reshape(n // GROUP, GROUP)
    sidx = scores_ids.astype(jnp.int32).reshape(n)
    out = _sc_embed(n, scores_emb.shape[0], words_emb, scores_emb,
                    widx2d, sidx)
    return out.reshape(b, l, D)


# speedup vs baseline: 3.3434x; 3.3434x over previous
import functools

import jax
import jax.numpy as jnp
from jax import lax
from jax.experimental import pallas as pl
from jax.experimental.pallas import tpu as pltpu
from jax.experimental.pallas import tpu_sc as plsc

NC = 2
NS = 16
NW = NC * NS

D = 64
R = 512
GROUP = 128
G = R // GROUP
NBUF = 2


def _sc_embed(n_rows, n_scores, words_emb, scores_emb, widx2d, sidx2d):
    per_w = n_rows // NW
    n_chunks = per_w // R

    mesh = plsc.VectorSubcoreMesh(
        core_axis_name="c", subcore_axis_name="s",
        num_cores=NC, num_subcores=NS,
    )

    @functools.partial(
        pl.kernel,
        out_type=jax.ShapeDtypeStruct((n_rows, D), jnp.float32),
        mesh=mesh,
        scratch_types=[
            pltpu.VMEM_SHARED((n_scores, D), jnp.float32),
            pltpu.VMEM((NBUF, G, GROUP), jnp.int32),
            pltpu.VMEM((NBUF, G, GROUP), jnp.int32),
            pltpu.VMEM((NBUF, R, D), jnp.float32),
            [pltpu.SemaphoreType.DMA] * NBUF,
            [pltpu.SemaphoreType.DMA] * NBUF,
            [pltpu.SemaphoreType.DMA] * NBUF,
            [pltpu.SemaphoreType.DMA] * NBUF,
        ],
        compiler_params=pltpu.CompilerParams(
            needs_layout_passes=False, use_tc_tiling_on_sc=False),
    )
    def k(words_hbm, stab_hbm, widx_hbm, sidx_hbm, out_hbm,
          stab_sh, widx_v, sidx_v, rows_v, semI, semW, semS, semO):
        cid = lax.axis_index("c")
        sid = lax.axis_index("s")
        wid = sid * NC + cid
        base_row = wid * per_w
        base_grp = wid * (per_w // GROUP)

        @pl.when(sid == 0)
        def _():
            pltpu.sync_copy(stab_hbm, stab_sh)

        plsc.subcore_barrier()

        dI = {b: [] for b in range(NBUF)}
        dW = {b: [] for b in range(NBUF)}
        dS = {b: [] for b in range(NBUF)}
        dO = {b: [] for b in range(NBUF)}

        def drain(descs):
            for d in descs:
                d.wait()
            descs.clear()

        def fire_idx(c):
            b = c % NBUF
            grp0 = base_grp + c * G
            dI[b].append(pltpu.async_copy(
                widx_hbm.at[pl.ds(grp0, G)], widx_v.at[b], semI[b]))
            dI[b].append(pltpu.async_copy(
                sidx_hbm.at[pl.ds(grp0, G)], sidx_v.at[b], semI[b]))

        def fire_words(c):
            b = c % NBUF
            for j in range(G):
                dW[b].append(pltpu.async_copy(
                    words_hbm.at[widx_v.at[b, j]],
                    rows_v.at[b, pl.ds(j * GROUP, GROUP)],
                    semW[b]))

        def fire_scores(c):
            b = c % NBUF
            for j in range(G):
                dS[b].append(pltpu.async_copy(
                    stab_sh.at[sidx_v.at[b, j]],
                    rows_v.at[b, pl.ds(j * GROUP, GROUP)],
                    semS[b], add=True))

        def fire_out(c):
            b = c % NBUF
            row0 = base_row + c * R
            dO[b].append(pltpu.async_copy(
                rows_v.at[b], out_hbm.at[pl.ds(row0, R)], semO[b]))

        fire_idx(0)
        for c in range(n_chunks):
            b = c % NBUF
            drain(dO[b])
            drain(dI[b])
            fire_words(c)
            if c == 0:
                fire_idx(1)
            else:
                p = c - 1
                d = p % NBUF
                drain(dW[d])
                fire_scores(p)
                drain(dS[d])
                if c + 1 < n_chunks:
                    fire_idx(c + 1)
                fire_out(p)
        p = n_chunks - 1
        d = p % NBUF
        drain(dW[d])
        fire_scores(p)
        drain(dS[d])
        fire_out(p)
        for b in range(NBUF):
            drain(dO[b])

    return k(words_emb, scores_emb, widx2d, sidx2d)


def kernel(input_ids, scores_ids, words_emb, scores_emb):
    b, l = input_ids.shape
    n = b * l
    widx2d = input_ids.astype(jnp.int32).reshape(n // GROUP, GROUP)
    sidx2d = scores_ids.astype(jnp.int32).reshape(n // GROUP, GROUP)
    out = _sc_embed(n, scores_emb.shape[0], words_emb, scores_emb,
                    widx2d, sidx2d)
    return out.reshape(b, l, D)


# speedup vs baseline: 3.3612x; 1.0053x over previous
import functools

import jax
import jax.numpy as jnp
from jax import lax
from jax.experimental import pallas as pl
from jax.experimental.pallas import tpu as pltpu
from jax.experimental.pallas import tpu_sc as plsc

NC = 2
NS = 16
NW = NC * NS

D = 64
M = 4
SEG = 128
NBUF = 2


def _sc_embed(batch, seq, n_scores, words_emb, scores_emb, widx, sidx):
    per_w = batch // NW
    n_chunks = per_w // M
    rem = seq - SEG

    mesh = plsc.VectorSubcoreMesh(
        core_axis_name="c", subcore_axis_name="s",
        num_cores=NC, num_subcores=NS,
    )

    @functools.partial(
        pl.kernel,
        out_type=jax.ShapeDtypeStruct((batch, seq, D), jnp.float32),
        mesh=mesh,
        scratch_types=[
            pltpu.VMEM_SHARED((n_scores, D), jnp.float32),
            pltpu.VMEM((NBUF, M, seq), jnp.int32),
            pltpu.VMEM((NBUF, M, seq), jnp.int32),
            pltpu.VMEM((NBUF, M, seq, D), jnp.float32),
            [pltpu.SemaphoreType.DMA] * NBUF,
            [pltpu.SemaphoreType.DMA] * NBUF,
            [pltpu.SemaphoreType.DMA] * NBUF,
            [pltpu.SemaphoreType.DMA] * NBUF,
        ],
        compiler_params=pltpu.CompilerParams(
            needs_layout_passes=False, use_tc_tiling_on_sc=False),
    )
    def k(words_hbm, stab_hbm, widx_hbm, sidx_hbm, out_hbm,
          stab_sh, widx_v, sidx_v, rows_v, semI, semW, semS, semO):
        cid = lax.axis_index("c")
        sid = lax.axis_index("s")
        wid = sid * NC + cid
        base_b = wid * per_w

        @pl.when(sid == 0)
        def _():
            pltpu.sync_copy(stab_hbm, stab_sh)

        plsc.subcore_barrier()

        dI = {b: [] for b in range(NBUF)}
        dW = {b: [] for b in range(NBUF)}
        dS = {b: [] for b in range(NBUF)}
        dO = {b: [] for b in range(NBUF)}

        def drain(descs):
            for d in descs:
                d.wait()
            descs.clear()

        def fire_idx(c):
            b = c % NBUF
            b0 = base_b + c * M
            dI[b].append(pltpu.async_copy(
                widx_hbm.at[pl.ds(b0, M)], widx_v.at[b], semI[b]))
            dI[b].append(pltpu.async_copy(
                sidx_hbm.at[pl.ds(b0, M)], sidx_v.at[b], semI[b]))

        def fire_gathers(c, idx_ref, src, sem_list, descs, add):
            b = c % NBUF
            for i in range(M):
                for off, ln in ((0, SEG), (SEG, rem)):
                    descs[b].append(pltpu.async_copy(
                        src.at[idx_ref.at[b, i, pl.ds(off, ln)]],
                        rows_v.at[b, i, pl.ds(off, ln)],
                        sem_list[b], add=add))

        def fire_out(c):
            b = c % NBUF
            b0 = base_b + c * M
            dO[b].append(pltpu.async_copy(
                rows_v.at[b], out_hbm.at[pl.ds(b0, M)], semO[b]))

        fire_idx(0)
        for c in range(n_chunks):
            b = c % NBUF
            drain(dO[b])
            drain(dI[b])
            fire_gathers(c, widx_v, words_hbm, semW, dW, False)
            if c == 0:
                fire_idx(1)
            else:
                p = c - 1
                d = p % NBUF
                drain(dW[d])
                fire_gathers(p, sidx_v, stab_sh, semS, dS, True)
                drain(dS[d])
                if c + 1 < n_chunks:
                    fire_idx(c + 1)
                fire_out(p)
        p = n_chunks - 1
        d = p % NBUF
        drain(dW[d])
        fire_gathers(p, sidx_v, stab_sh, semS, dS, True)
        drain(dS[d])
        fire_out(p)
        for b in range(NBUF):
            drain(dO[b])

    return k(words_emb, scores_emb, widx, sidx)


def kernel(input_ids, scores_ids, words_emb, scores_emb):
    batch, seq = input_ids.shape
    return _sc_embed(batch, seq, scores_emb.shape[0],
                     words_emb, scores_emb,
                     input_ids.astype(jnp.int32),
                     scores_ids.astype(jnp.int32))


# speedup vs baseline: 4.0790x; 1.2136x over previous
import functools

import jax
import jax.numpy as jnp
from jax import lax
from jax.experimental import pallas as pl
from jax.experimental.pallas import tpu as pltpu
from jax.experimental.pallas import tpu_sc as plsc

NC = 2
NS = 16
NW = NC * NS

D = 64
DP = 128
M = 2
SEG = 128
NBUF = 2


def _sc_embed(batch, seq, n_scores, words_pad, scores_pad, widx, sidx):
    per_w = batch // NW
    n_chunks = per_w // M
    rem = seq - SEG

    mesh = plsc.VectorSubcoreMesh(
        core_axis_name="c", subcore_axis_name="s",
        num_cores=NC, num_subcores=NS,
    )

    @functools.partial(
        pl.kernel,
        out_type=jax.ShapeDtypeStruct((batch, seq, DP), jnp.float32),
        mesh=mesh,
        scratch_types=[
            pltpu.VMEM_SHARED((n_scores, DP), jnp.float32),
            pltpu.VMEM((NBUF, M, seq), jnp.int32),
            pltpu.VMEM((NBUF, M, seq), jnp.int32),
            pltpu.VMEM((NBUF, M, seq, DP), jnp.float32),
            [pltpu.SemaphoreType.DMA] * NBUF,
            [pltpu.SemaphoreType.DMA] * NBUF,
            [pltpu.SemaphoreType.DMA] * NBUF,
            [pltpu.SemaphoreType.DMA] * NBUF,
        ],
        compiler_params=pltpu.CompilerParams(
            needs_layout_passes=False, use_tc_tiling_on_sc=True),
    )
    def k(words_hbm, stab_hbm, widx_hbm, sidx_hbm, out_hbm,
          stab_sh, widx_v, sidx_v, rows_v, semI, semW, semS, semO):
        cid = lax.axis_index("c")
        sid = lax.axis_index("s")
        wid = sid * NC + cid
        base_b = wid * per_w

        @pl.when(sid == 0)
        def _():
            pltpu.sync_copy(stab_hbm, stab_sh)

        plsc.subcore_barrier()

        dI = {b: [] for b in range(NBUF)}
        dW = {b: [] for b in range(NBUF)}
        dS = {b: [] for b in range(NBUF)}
        dO = {b: [] for b in range(NBUF)}

        def drain(descs):
            for d in descs:
                d.wait()
            descs.clear()

        def fire_idx(c):
            b = c % NBUF
            b0 = base_b + c * M
            dI[b].append(pltpu.async_copy(
                widx_hbm.at[pl.ds(b0, M)], widx_v.at[b], semI[b]))
            dI[b].append(pltpu.async_copy(
                sidx_hbm.at[pl.ds(b0, M)], sidx_v.at[b], semI[b]))

        def fire_gathers(c, idx_ref, src, sem_list, descs, add):
            b = c % NBUF
            for i in range(M):
                for off, ln in ((0, SEG), (SEG, rem)):
                    descs[b].append(pltpu.async_copy(
                        src.at[idx_ref.at[b, i, pl.ds(off, ln)]],
                        rows_v.at[b, i, pl.ds(off, ln)],
                        sem_list[b], add=add))

        def fire_out(c):
            b = c % NBUF
            b0 = base_b + c * M
            dO[b].append(pltpu.async_copy(
                rows_v.at[b], out_hbm.at[pl.ds(b0, M)], semO[b]))

        fire_idx(0)
        for c in range(n_chunks):
            b = c % NBUF
            drain(dO[b])
            drain(dI[b])
            fire_gathers(c, widx_v, words_hbm, semW, dW, False)
            if c == 0:
                fire_idx(1)
            else:
                p = c - 1
                d = p % NBUF
                drain(dW[d])
                fire_gathers(p, sidx_v, stab_sh, semS, dS, True)
                drain(dS[d])
                if c + 1 < n_chunks:
                    fire_idx(c + 1)
                fire_out(p)
        p = n_chunks - 1
        d = p % NBUF
        drain(dW[d])
        fire_gathers(p, sidx_v, stab_sh, semS, dS, True)
        drain(dS[d])
        fire_out(p)
        for b in range(NBUF):
            drain(dO[b])

    return k(words_pad, scores_pad, widx, sidx)


def kernel(input_ids, scores_ids, words_emb, scores_emb):
    batch, seq = input_ids.shape
    words_pad = jnp.pad(words_emb, ((0, 0), (0, DP - D)))
    scores_pad = jnp.pad(scores_emb, ((0, 0), (0, DP - D)))
    out_pad = _sc_embed(batch, seq, scores_emb.shape[0],
                        words_pad, scores_pad,
                        input_ids.astype(jnp.int32),
                        scores_ids.astype(jnp.int32))
    return out_pad[:, :, :D]
